# baseline (speedup 1.0000x reference)
import functools

import jax
import jax.numpy as jnp
from jax import lax
from jax.experimental import pallas as pl
from jax.experimental.pallas import tpu as pltpu
from jax.experimental.pallas import tpu_sc as plsc

N = 10000
E = 320000
D = 128
G = 64

DP = 144
NC, NS, L = 2, 16, 16
NW = NC * NS
EPT = E // NW
K = 80
NCHUNK = EPT // K
NP = 10240
RPT = NP // NS

_f32 = jnp.float32


def _attn_tail(h, asrc_ref, adst_ref, ht_ref, ad_ref):
    a_s = jnp.sum(h * asrc_ref[...], axis=1, keepdims=True)
    ht_ref[...] = jnp.concatenate(
        [h, jnp.ones((N, 1), _f32), a_s, jnp.zeros((N, DP - D - 2), _f32)],
        axis=1)
    ad_ref[...] = jnp.sum(h * adst_ref[...], axis=1, keepdims=True)


def _head_body(x_ref, w_ref, asrc_ref, adst_ref, ht_ref, ad_ref):
    h = jnp.dot(x_ref[...], w_ref[...], preferred_element_type=_f32,
                precision=lax.Precision.HIGHEST)
    _attn_tail(h, asrc_ref, adst_ref, ht_ref, ad_ref)


def _combine_bn_relu(p_ref, b_ref, g_ref, beta_ref):
    s = p_ref[0, :N] + p_ref[1, :N]
    z = s[:, D:D + 1]
    out = s[:, :D] / (z + 1e-16) + b_ref[...]
    mu = jnp.mean(out, axis=0, keepdims=True)
    var = jnp.mean((out - mu) ** 2, axis=0, keepdims=True)
    y = (out - mu) * lax.rsqrt(var + 1e-5) * g_ref[...] + beta_ref[...]
    return jnp.maximum(y, 0.0)


def _mid_body(p_ref, b_ref, g_ref, beta_ref, w_ref, asrc_ref, adst_ref,
              ht_ref, ad_ref):
    y = _combine_bn_relu(p_ref, b_ref, g_ref, beta_ref)
    h = jnp.dot(y, w_ref[...], preferred_element_type=_f32,
                precision=lax.Precision.HIGHEST)
    _attn_tail(h, asrc_ref, adst_ref, ht_ref, ad_ref)


def _tail_body(p_ref, b_ref, g_ref, beta_ref, batch_ref, l1w_ref, l1b_ref,
               l2w_ref, l2b_ref, out_ref):
    y = _combine_bn_relu(p_ref, b_ref, g_ref, beta_ref)
    gids = lax.broadcasted_iota(jnp.int32, (G, N), 0)
    onehot = (jnp.broadcast_to(batch_ref[...], (G, N)) == gids).astype(_f32)
    sums = jnp.dot(onehot, y, preferred_element_type=_f32,
                   precision=lax.Precision.HIGHEST)
    cnt = jnp.sum(onehot, axis=1, keepdims=True)
    gfeat = sums / jnp.maximum(cnt, 1.0)
    gfeat = jnp.maximum(
        jnp.dot(gfeat, l1w_ref[...], preferred_element_type=_f32,
                precision=lax.Precision.HIGHEST)
        + l1b_ref[...], 0.0)
    out_ref[...] = (jnp.dot(gfeat, l2w_ref[...], preferred_element_type=_f32,
                            precision=lax.Precision.HIGHEST)
                    + l2b_ref[...])


_tc_params = pltpu.CompilerParams(vmem_limit_bytes=100 * 1024 * 1024)

_head = pl.pallas_call(
    _head_body,
    out_shape=(jax.ShapeDtypeStruct((N, DP), _f32),
               jax.ShapeDtypeStruct((N, 1), _f32)),
    compiler_params=_tc_params)

_mid = pl.pallas_call(
    _mid_body,
    out_shape=(jax.ShapeDtypeStruct((N, DP), _f32),
               jax.ShapeDtypeStruct((N, 1), _f32)),
    compiler_params=_tc_params)

_tail = pl.pallas_call(
    _tail_body,
    out_shape=jax.ShapeDtypeStruct((G, 1), _f32),
    compiler_params=_tc_params)


CPB = 25
BLK = CPB * K


def _sc_edge_body(ht_hbm, ad_hbm, src_hbm, dst_hbm, zeros_hbm, out_hbm,
                  adv, sblk, dblk, d0, d1, r0, r1, acc, g0, g1, c0, c1):
    cid = lax.axis_index("c")
    t = lax.axis_index("s")
    Dd = (d0, d1)
    R = (r0, r1)
    Gs = (g0, g1)
    Cs = (c0, c1)

    pltpu.sync_copy(ad_hbm, adv)

    pltpu.sync_copy(zeros_hbm, r0)
    for r in range(RPT // K):
        pltpu.sync_copy(r0, acc.at[pl.ds(t * RPT + r * K, K)])
    plsc.subcore_barrier()

    ebase = (cid * NS + t) * EPT

    def load_block(iblk):
        off = ebase + iblk * BLK
        pltpu.sync_copy(src_hbm.at[pl.ds(off, BLK)], sblk)
        pltpu.sync_copy(dst_hbm.at[pl.ds(off, BLK)], dblk)

    def issue_gather(ib, b):
        cb = lax.rem(ib, CPB) * K
        for v in range(K // L):
            Dd[b][pl.ds(v * L, L)] = dblk[pl.ds(cb + v * L, L)]
        pltpu.async_copy(ht_hbm.at[sblk.at[pl.ds(cb, K)]], R[b], Gs[b])

    def step(ib, b):
        nb = 1 - b
        @pl.when(ib >= 1)
        def _():
            pltpu.make_async_copy(zeros_hbm, R[nb], Cs[nb]).wait()

        blockstart = lax.rem(ib + 1, CPB) == 0

        @pl.when(blockstart)
        def _():
            pltpu.make_async_copy(ht_hbm.at[sblk.at[pl.ds(0, K)]],
                                  R[b], Gs[b]).wait()

            @pl.when(ib + 1 < NCHUNK)
            def _():
                load_block((ib + 1) // CPB)
                issue_gather(ib + 1, nb)

        @pl.when(jnp.logical_not(blockstart))
        def _():
            issue_gather(ib + 1, nb)
            pltpu.make_async_copy(ht_hbm.at[sblk.at[pl.ds(0, K)]],
                                  R[b], Gs[b]).wait()

        rows = R[b]
        dstv = Dd[b]

        for jg in range(K // L):
            di = dstv[pl.ds(jg * L, L)]
            advec = plsc.load_gather(adv, [di])
            rowid = jg * L + lax.iota(jnp.int32, L)
            asvec = plsc.load_gather(
                rows, [rowid, jnp.full((L,), D + 1, jnp.int32)])
            e = asvec + advec
            e = jnp.where(e >= 0.0, e, 0.2 * e)
            wvec = jnp.exp(e)
            plsc.store_scatter(rows, [rowid, jnp.full((L,), D, jnp.int32)],
                               wvec)
            for jj in range(L):
                wj = wvec[jj]
                row = jg * L + jj
                for v in range(D // L):
                    sl = pl.ds(v * L, L)
                    rows[row, sl] = rows[row, sl] * wj

        pltpu.async_copy(rows, acc.at[dstv], Cs[b], add=True)

    load_block(0)
    issue_gather(0, 0)

    def pair(ip, carry):
        step(2 * ip, 0)
        step(2 * ip + 1, 1)
        return carry
    lax.fori_loop(0, NCHUNK // 2, pair, 0)
    step(NCHUNK - 1, 0)

    pltpu.make_async_copy(zeros_hbm, R[0], Cs[0]).wait()
    plsc.subcore_barrier()

    for r in range(RPT // K):
        base = t * RPT + r * K
        pltpu.sync_copy(acc.at[pl.ds(base, K)], r0)
        pltpu.sync_copy(r0, out_hbm.at[cid, pl.ds(base, K)])


_sc_edge = pl.kernel(
    _sc_edge_body,
    out_type=jax.ShapeDtypeStruct((NC, NP, DP), _f32),
    mesh=plsc.VectorSubcoreMesh(core_axis_name="c", subcore_axis_name="s"),
    compiler_params=pltpu.CompilerParams(use_tc_tiling_on_sc=False,
                                         needs_layout_passes=False),
    scratch_types=[
        pltpu.VMEM((N,), _f32),
        pltpu.VMEM((BLK,), jnp.int32),
        pltpu.VMEM((BLK,), jnp.int32),
        pltpu.VMEM((K,), jnp.int32),
        pltpu.VMEM((K,), jnp.int32),
        pltpu.VMEM((K, DP), _f32),
        pltpu.VMEM((K, DP), _f32),
        pltpu.VMEM_SHARED((NP, DP), _f32),
        pltpu.SemaphoreType.DMA,
        pltpu.SemaphoreType.DMA,
        pltpu.SemaphoreType.DMA,
        pltpu.SemaphoreType.DMA,
    ])


def kernel(x, edge_index, batch, params):
    src = edge_index[0]
    dst = edge_index[1]
    zeros = jnp.zeros((K, DP), _f32)

    p1, p2, p3 = params["gat1"], params["gat2"], params["gat3"]
    bn1, bn2, bn3 = params["bn1"], params["bn2"], params["bn3"]

    ht, a_d = _head(x, p1["W"], p1["a_src"], p1["a_dst"])
    part = _sc_edge(ht, a_d.reshape(N), src, dst, zeros)
    ht, a_d = _mid(part, p1["b"], bn1["g"], bn1["b"],
                   p2["W"], p2["a_src"], p2["a_dst"])
    part = _sc_edge(ht, a_d.reshape(N), src, dst, zeros)
    ht, a_d = _mid(part, p2["b"], bn2["g"], bn2["b"],
                   p3["W"], p3["a_src"], p3["a_dst"])
    part = _sc_edge(ht, a_d.reshape(N), src, dst, zeros)
    return _tail(part, p3["b"], bn3["g"], bn3["b"], batch.reshape(1, N),
                 params["lin1_W"], params["lin1_b"],
                 params["lin2_W"], params["lin2_b"])


# speedup vs baseline: 1.2141x; 1.2141x over previous
import functools

import jax
import jax.numpy as jnp
from jax import lax
from jax.experimental import pallas as pl
from jax.experimental.pallas import tpu as pltpu
from jax.experimental.pallas import tpu_sc as plsc

N = 10000
E = 320000
D = 128
G = 64

DP = 144
NC, NS, L = 2, 16, 16
NW = NC * NS
EPT = E // NW
K = 80
NCHUNK = EPT // K
NP = 10240
RPT = NP // NS

_f32 = jnp.float32


def _attn_tail(h, asrc_ref, adst_ref, ht_ref, ad_ref):
    a_s = jnp.sum(h * asrc_ref[...], axis=1, keepdims=True)
    ht_ref[...] = jnp.concatenate(
        [h, jnp.ones((N, 1), _f32), a_s, jnp.zeros((N, DP - D - 2), _f32)],
        axis=1)
    ad_ref[...] = jnp.sum(h * adst_ref[...], axis=1, keepdims=True)


def _head_body(x_ref, w_ref, asrc_ref, adst_ref, ht_ref, ad_ref):
    h = jnp.dot(x_ref[...], w_ref[...], preferred_element_type=_f32,
                precision=lax.Precision.HIGHEST)
    _attn_tail(h, asrc_ref, adst_ref, ht_ref, ad_ref)


def _combine_bn_relu(p_ref, b_ref, g_ref, beta_ref):
    s = p_ref[0, :N] + p_ref[1, :N]
    z = s[:, D:D + 1]
    out = s[:, :D] / (z + 1e-16) + b_ref[...]
    mu = jnp.mean(out, axis=0, keepdims=True)
    var = jnp.mean((out - mu) ** 2, axis=0, keepdims=True)
    y = (out - mu) * lax.rsqrt(var + 1e-5) * g_ref[...] + beta_ref[...]
    return jnp.maximum(y, 0.0)


def _mid_body(p_ref, b_ref, g_ref, beta_ref, w_ref, asrc_ref, adst_ref,
              ht_ref, ad_ref):
    y = _combine_bn_relu(p_ref, b_ref, g_ref, beta_ref)
    h = jnp.dot(y, w_ref[...], preferred_element_type=_f32,
                precision=lax.Precision.HIGHEST)
    _attn_tail(h, asrc_ref, adst_ref, ht_ref, ad_ref)


def _tail_body(p_ref, b_ref, g_ref, beta_ref, batch_ref, l1w_ref, l1b_ref,
               l2w_ref, l2b_ref, out_ref):
    y = _combine_bn_relu(p_ref, b_ref, g_ref, beta_ref)
    gids = lax.broadcasted_iota(jnp.int32, (G, N), 0)
    onehot = (jnp.broadcast_to(batch_ref[...], (G, N)) == gids).astype(_f32)
    sums = jnp.dot(onehot, y, preferred_element_type=_f32,
                   precision=lax.Precision.HIGHEST)
    cnt = jnp.sum(onehot, axis=1, keepdims=True)
    gfeat = sums / jnp.maximum(cnt, 1.0)
    gfeat = jnp.maximum(
        jnp.dot(gfeat, l1w_ref[...], preferred_element_type=_f32,
                precision=lax.Precision.HIGHEST)
        + l1b_ref[...], 0.0)
    out_ref[...] = (jnp.dot(gfeat, l2w_ref[...], preferred_element_type=_f32,
                            precision=lax.Precision.HIGHEST)
                    + l2b_ref[...])


_tc_params = pltpu.CompilerParams(vmem_limit_bytes=100 * 1024 * 1024)

_head = pl.pallas_call(
    _head_body,
    out_shape=(jax.ShapeDtypeStruct((N, DP), _f32),
               jax.ShapeDtypeStruct((N, 1), _f32)),
    compiler_params=_tc_params)

_mid = pl.pallas_call(
    _mid_body,
    out_shape=(jax.ShapeDtypeStruct((N, DP), _f32),
               jax.ShapeDtypeStruct((N, 1), _f32)),
    compiler_params=_tc_params)

_tail = pl.pallas_call(
    _tail_body,
    out_shape=jax.ShapeDtypeStruct((G, 1), _f32),
    compiler_params=_tc_params)


CPB = 25
BLK = CPB * K


def _sc_edge_body(ht_hbm, ad_hbm, src_hbm, dst_hbm, zeros_hbm, out_hbm,
                  adv, sblk, dblk, d0, d1, r0, r1, acc, g0, g1, c0, c1):
    cid = lax.axis_index("c")
    t = lax.axis_index("s")
    Dd = (d0, d1)
    R = (r0, r1)
    Gs = (g0, g1)
    Cs = (c0, c1)

    pltpu.sync_copy(ad_hbm, adv)

    pltpu.sync_copy(zeros_hbm, r0)
    for r in range(RPT // K):
        pltpu.sync_copy(r0, acc.at[pl.ds(t * RPT + r * K, K)])
    plsc.subcore_barrier()

    ebase = (cid * NS + t) * EPT

    def load_block(iblk):
        off = ebase + iblk * BLK
        pltpu.sync_copy(src_hbm.at[pl.ds(off, BLK)], sblk)
        pltpu.sync_copy(dst_hbm.at[pl.ds(off, BLK)], dblk)

    def issue_gather(ib, b):
        cb = lax.rem(ib, CPB) * K
        for v in range(K // L):
            Dd[b][pl.ds(v * L, L)] = dblk[pl.ds(cb + v * L, L)]
        pltpu.async_copy(ht_hbm.at[sblk.at[pl.ds(cb, K)]], R[b], Gs[b])

    def step(ib, b):
        nb = 1 - b
        @pl.when(ib >= 1)
        def _():
            pltpu.make_async_copy(zeros_hbm, R[nb], Cs[nb]).wait()

        blockstart = lax.rem(ib + 1, CPB) == 0

        @pl.when(blockstart)
        def _():
            pltpu.make_async_copy(ht_hbm.at[sblk.at[pl.ds(0, K)]],
                                  R[b], Gs[b]).wait()

            @pl.when(ib + 1 < NCHUNK)
            def _():
                load_block((ib + 1) // CPB)
                issue_gather(ib + 1, nb)

        @pl.when(jnp.logical_not(blockstart))
        def _():
            issue_gather(ib + 1, nb)
            pltpu.make_async_copy(ht_hbm.at[sblk.at[pl.ds(0, K)]],
                                  R[b], Gs[b]).wait()

        rows = R[b]
        dstv = Dd[b]

        @plsc.parallel_loop(0, K // L)
        def grp(jg):
            di = dstv[pl.ds(jg * L, L)]
            advec = plsc.load_gather(adv, [di])
            rowid = jg * L + lax.iota(jnp.int32, L)
            asvec = plsc.load_gather(
                rows, [rowid, jnp.full((L,), D + 1, jnp.int32)])
            e = asvec + advec
            e = jnp.where(e >= 0.0, e, 0.2 * e)
            wvec = jnp.exp(e)
            plsc.store_scatter(rows, [rowid, jnp.full((L,), D, jnp.int32)],
                               wvec)
            for jj in range(L):
                wj = wvec[jj]
                row = jg * L + jj
                for v in range(D // L):
                    sl = pl.ds(v * L, L)
                    rows[row, sl] = rows[row, sl] * wj

        pltpu.async_copy(rows, acc.at[dstv], Cs[b], add=True)

    load_block(0)
    issue_gather(0, 0)

    def pair(ip, carry):
        step(2 * ip, 0)
        step(2 * ip + 1, 1)
        return carry
    lax.fori_loop(0, NCHUNK // 2, pair, 0)
    step(NCHUNK - 1, 0)

    pltpu.make_async_copy(zeros_hbm, R[0], Cs[0]).wait()
    plsc.subcore_barrier()

    for r in range(RPT // K):
        base = t * RPT + r * K
        pltpu.sync_copy(acc.at[pl.ds(base, K)], r0)
        pltpu.sync_copy(r0, out_hbm.at[cid, pl.ds(base, K)])


_sc_edge = pl.kernel(
    _sc_edge_body,
    out_type=jax.ShapeDtypeStruct((NC, NP, DP), _f32),
    mesh=plsc.VectorSubcoreMesh(core_axis_name="c", subcore_axis_name="s"),
    compiler_params=pltpu.CompilerParams(use_tc_tiling_on_sc=False,
                                         needs_layout_passes=False),
    scratch_types=[
        pltpu.VMEM((N,), _f32),
        pltpu.VMEM((BLK,), jnp.int32),
        pltpu.VMEM((BLK,), jnp.int32),
        pltpu.VMEM((K,), jnp.int32),
        pltpu.VMEM((K,), jnp.int32),
        pltpu.VMEM((K, DP), _f32),
        pltpu.VMEM((K, DP), _f32),
        pltpu.VMEM_SHARED((NP, DP), _f32),
        pltpu.SemaphoreType.DMA,
        pltpu.SemaphoreType.DMA,
        pltpu.SemaphoreType.DMA,
        pltpu.SemaphoreType.DMA,
    ])


def kernel(x, edge_index, batch, params):
    src = edge_index[0]
    dst = edge_index[1]
    zeros = jnp.zeros((K, DP), _f32)

    p1, p2, p3 = params["gat1"], params["gat2"], params["gat3"]
    bn1, bn2, bn3 = params["bn1"], params["bn2"], params["bn3"]

    ht, a_d = _head(x, p1["W"], p1["a_src"], p1["a_dst"])
    part = _sc_edge(ht, a_d.reshape(N), src, dst, zeros)
    ht, a_d = _mid(part, p1["b"], bn1["g"], bn1["b"],
                   p2["W"], p2["a_src"], p2["a_dst"])
    part = _sc_edge(ht, a_d.reshape(N), src, dst, zeros)
    ht, a_d = _mid(part, p2["b"], bn2["g"], bn2["b"],
                   p3["W"], p3["a_src"], p3["a_dst"])
    part = _sc_edge(ht, a_d.reshape(N), src, dst, zeros)
    return _tail(part, p3["b"], bn3["g"], bn3["b"], batch.reshape(1, N),
                 params["lin1_W"], params["lin1_b"],
                 params["lin2_W"], params["lin2_b"])


# speedup vs baseline: 1.2143x; 1.0002x over previous
import functools

import jax
import jax.numpy as jnp
from jax import lax
from jax.experimental import pallas as pl
from jax.experimental.pallas import tpu as pltpu
from jax.experimental.pallas import tpu_sc as plsc

N = 10000
E = 320000
D = 128
G = 64

DP = 144
NC, NS, L = 2, 16, 16
NW = NC * NS
EPT = E // NW
K = 80
NCHUNK = EPT // K
NP = 10240
RPT = NP // NS

_f32 = jnp.float32


def _dot3(x, w):
    return jnp.dot(x, w, preferred_element_type=_f32,
                   precision=lax.Precision.HIGHEST)


def _attn_tail(h, asrc_ref, adst_ref, ht_ref, ad_ref):
    a_s = jnp.sum(h * asrc_ref[...], axis=1, keepdims=True)
    ht_ref[...] = jnp.concatenate(
        [h, jnp.ones((N, 1), _f32), a_s, jnp.zeros((N, DP - D - 2), _f32)],
        axis=1)
    ad_ref[...] = jnp.sum(h * adst_ref[...], axis=1, keepdims=True)


def _head_body(x_ref, w_ref, asrc_ref, adst_ref, ht_ref, ad_ref):
    h = _dot3(x_ref[...], w_ref[...])
    _attn_tail(h, asrc_ref, adst_ref, ht_ref, ad_ref)


def _combine_bn_relu(p_ref, b_ref, g_ref, beta_ref):
    s = p_ref[0, :N] + p_ref[1, :N]
    z = s[:, D:D + 1]
    out = s[:, :D] / (z + 1e-16) + b_ref[...]
    mu = jnp.mean(out, axis=0, keepdims=True)
    var = jnp.mean((out - mu) ** 2, axis=0, keepdims=True)
    y = (out - mu) * lax.rsqrt(var + 1e-5) * g_ref[...] + beta_ref[...]
    return jnp.maximum(y, 0.0)


def _mid_body(p_ref, b_ref, g_ref, beta_ref, w_ref, asrc_ref, adst_ref,
              ht_ref, ad_ref):
    y = _combine_bn_relu(p_ref, b_ref, g_ref, beta_ref)
    h = _dot3(y, w_ref[...])
    _attn_tail(h, asrc_ref, adst_ref, ht_ref, ad_ref)


def _tail_body(p_ref, b_ref, g_ref, beta_ref, batch_ref, l1w_ref, l1b_ref,
               l2w_ref, l2b_ref, out_ref):
    y = _combine_bn_relu(p_ref, b_ref, g_ref, beta_ref)
    gids = lax.broadcasted_iota(jnp.int32, (G, N), 0)
    onehot = (jnp.broadcast_to(batch_ref[...], (G, N)) == gids).astype(_f32)
    sums = _dot3(onehot, y)
    cnt = jnp.sum(onehot, axis=1, keepdims=True)
    gfeat = sums / jnp.maximum(cnt, 1.0)
    gfeat = jnp.maximum(
        _dot3(gfeat, l1w_ref[...])
        + l1b_ref[...], 0.0)
    out_ref[...] = (_dot3(gfeat, l2w_ref[...])
                    + l2b_ref[...])


_tc_params = pltpu.CompilerParams(vmem_limit_bytes=100 * 1024 * 1024)

_head = pl.pallas_call(
    _head_body,
    out_shape=(jax.ShapeDtypeStruct((N, DP), _f32),
               jax.ShapeDtypeStruct((N, 1), _f32)),
    compiler_params=_tc_params)

_mid = pl.pallas_call(
    _mid_body,
    out_shape=(jax.ShapeDtypeStruct((N, DP), _f32),
               jax.ShapeDtypeStruct((N, 1), _f32)),
    compiler_params=_tc_params)

_tail = pl.pallas_call(
    _tail_body,
    out_shape=jax.ShapeDtypeStruct((G, 1), _f32),
    compiler_params=_tc_params)


CPB = 25
BLK = CPB * K


def _sc_edge_body(ht_hbm, ad_hbm, src_hbm, dst_hbm, zeros_hbm, out_hbm,
                  adv, sblk, dblk, d0, d1, r0, r1, acc, g0, g1, c0, c1):
    cid = lax.axis_index("c")
    t = lax.axis_index("s")
    Dd = (d0, d1)
    R = (r0, r1)
    Gs = (g0, g1)
    Cs = (c0, c1)

    pltpu.sync_copy(ad_hbm, adv)

    pltpu.sync_copy(zeros_hbm, r0)
    for r in range(RPT // K):
        pltpu.sync_copy(r0, acc.at[pl.ds(t * RPT + r * K, K)])
    plsc.subcore_barrier()

    ebase = (cid * NS + t) * EPT

    def load_block(iblk):
        off = ebase + iblk * BLK
        pltpu.sync_copy(src_hbm.at[pl.ds(off, BLK)], sblk)
        pltpu.sync_copy(dst_hbm.at[pl.ds(off, BLK)], dblk)

    def issue_gather(ib, b):
        cb = lax.rem(ib, CPB) * K
        for v in range(K // L):
            Dd[b][pl.ds(v * L, L)] = dblk[pl.ds(cb + v * L, L)]
        pltpu.async_copy(ht_hbm.at[sblk.at[pl.ds(cb, K)]], R[b], Gs[b])

    def step(ib, b):
        nb = 1 - b
        @pl.when(ib >= 1)
        def _():
            pltpu.make_async_copy(zeros_hbm, R[nb], Cs[nb]).wait()

        blockstart = lax.rem(ib + 1, CPB) == 0

        @pl.when(blockstart)
        def _():
            pltpu.make_async_copy(ht_hbm.at[sblk.at[pl.ds(0, K)]],
                                  R[b], Gs[b]).wait()

            @pl.when(ib + 1 < NCHUNK)
            def _():
                load_block((ib + 1) // CPB)
                issue_gather(ib + 1, nb)

        @pl.when(jnp.logical_not(blockstart))
        def _():
            issue_gather(ib + 1, nb)
            pltpu.make_async_copy(ht_hbm.at[sblk.at[pl.ds(0, K)]],
                                  R[b], Gs[b]).wait()

        rows = R[b]
        dstv = Dd[b]

        @plsc.parallel_loop(0, K // L)
        def grp(jg):
            di = dstv[pl.ds(jg * L, L)]
            advec = plsc.load_gather(adv, [di])
            rowid = jg * L + lax.iota(jnp.int32, L)
            asvec = plsc.load_gather(
                rows, [rowid, jnp.full((L,), D + 1, jnp.int32)])
            e = asvec + advec
            e = jnp.where(e >= 0.0, e, 0.2 * e)
            wvec = jnp.exp(e)
            plsc.store_scatter(rows, [rowid, jnp.full((L,), D, jnp.int32)],
                               wvec)
            for jj in range(L):
                wj = wvec[jj]
                row = jg * L + jj
                for v in range(D // L):
                    sl = pl.ds(v * L, L)
                    rows[row, sl] = rows[row, sl] * wj

        pltpu.async_copy(rows, acc.at[dstv], Cs[b], add=True)

    load_block(0)
    issue_gather(0, 0)

    def pair(ip, carry):
        step(2 * ip, 0)
        step(2 * ip + 1, 1)
        return carry
    lax.fori_loop(0, NCHUNK // 2, pair, 0)
    step(NCHUNK - 1, 0)

    pltpu.make_async_copy(zeros_hbm, R[0], Cs[0]).wait()
    plsc.subcore_barrier()

    for r in range(RPT // K):
        base = t * RPT + r * K
        pltpu.sync_copy(acc.at[pl.ds(base, K)], r0)
        pltpu.sync_copy(r0, out_hbm.at[cid, pl.ds(base, K)])


_sc_edge = pl.kernel(
    _sc_edge_body,
    out_type=jax.ShapeDtypeStruct((NC, NP, DP), _f32),
    mesh=plsc.VectorSubcoreMesh(core_axis_name="c", subcore_axis_name="s"),
    compiler_params=pltpu.CompilerParams(use_tc_tiling_on_sc=False,
                                         needs_layout_passes=False),
    scratch_types=[
        pltpu.VMEM((N,), _f32),
        pltpu.VMEM((BLK,), jnp.int32),
        pltpu.VMEM((BLK,), jnp.int32),
        pltpu.VMEM((K,), jnp.int32),
        pltpu.VMEM((K,), jnp.int32),
        pltpu.VMEM((K, DP), _f32),
        pltpu.VMEM((K, DP), _f32),
        pltpu.VMEM_SHARED((NP, DP), _f32),
        pltpu.SemaphoreType.DMA,
        pltpu.SemaphoreType.DMA,
        pltpu.SemaphoreType.DMA,
        pltpu.SemaphoreType.DMA,
    ])


def kernel(x, edge_index, batch, params):
    src = edge_index[0]
    dst = edge_index[1]
    zeros = jnp.zeros((K, DP), _f32)

    p1, p2, p3 = params["gat1"], params["gat2"], params["gat3"]
    bn1, bn2, bn3 = params["bn1"], params["bn2"], params["bn3"]

    ht, a_d = _head(x, p1["W"], p1["a_src"], p1["a_dst"])
    part = _sc_edge(ht, a_d.reshape(N), src, dst, zeros)
    ht, a_d = _mid(part, p1["b"], bn1["g"], bn1["b"],
                   p2["W"], p2["a_src"], p2["a_dst"])
    part = _sc_edge(ht, a_d.reshape(N), src, dst, zeros)
    ht, a_d = _mid(part, p2["b"], bn2["g"], bn2["b"],
                   p3["W"], p3["a_src"], p3["a_dst"])
    part = _sc_edge(ht, a_d.reshape(N), src, dst, zeros)
    return _tail(part, p3["b"], bn3["g"], bn3["b"], batch.reshape(1, N),
                 params["lin1_W"], params["lin1_b"],
                 params["lin2_W"], params["lin2_b"])


# speedup vs baseline: 1.2169x; 1.0022x over previous
import jax
import jax.numpy as jnp
from jax import lax
from jax.experimental import pallas as pl
from jax.experimental.pallas import tpu as pltpu
from jax.experimental.pallas import tpu_sc as plsc

N = 10000
E = 320000
D = 128
G = 64

DP = 144
NC, NS, L = 2, 16, 16
NW = NC * NS
EPT = E // NW
K = 80
NCHUNK = EPT // K
NP = 10240
RPT = NP // NS

_f32 = jnp.float32


def _dot_f32(x, w):
    return jnp.dot(x, w, preferred_element_type=_f32,
                   precision=lax.Precision.HIGHEST)


def _attn_tail(h, asrc_ref, adst_ref, ht_ref, ad_ref):
    a_s = jnp.sum(h * asrc_ref[...], axis=1, keepdims=True)
    ht_ref[...] = jnp.concatenate(
        [h, jnp.ones((N, 1), _f32), a_s, jnp.zeros((N, DP - D - 2), _f32)],
        axis=1)
    ad_ref[...] = jnp.sum(h * adst_ref[...], axis=1, keepdims=True)


def _head_body(x_ref, w_ref, asrc_ref, adst_ref, ht_ref, ad_ref):
    h = _dot_f32(x_ref[...], w_ref[...])
    _attn_tail(h, asrc_ref, adst_ref, ht_ref, ad_ref)


def _combine_bn_relu(p_ref, b_ref, g_ref, beta_ref):
    s = p_ref[0, :N] + p_ref[1, :N]
    z = s[:, D:D + 1]
    out = s[:, :D] / (z + 1e-16) + b_ref[...]
    mu = jnp.mean(out, axis=0, keepdims=True)
    var = jnp.mean((out - mu) ** 2, axis=0, keepdims=True)
    y = (out - mu) * lax.rsqrt(var + 1e-5) * g_ref[...] + beta_ref[...]
    return jnp.maximum(y, 0.0)


def _mid_body(p_ref, b_ref, g_ref, beta_ref, w_ref, asrc_ref, adst_ref,
              ht_ref, ad_ref):
    y = _combine_bn_relu(p_ref, b_ref, g_ref, beta_ref)
    h = _dot_f32(y, w_ref[...])
    _attn_tail(h, asrc_ref, adst_ref, ht_ref, ad_ref)


def _tail_body(p_ref, b_ref, g_ref, beta_ref, batch_ref, l1w_ref, l1b_ref,
               l2w_ref, l2b_ref, out_ref):
    y = _combine_bn_relu(p_ref, b_ref, g_ref, beta_ref)
    gids = lax.broadcasted_iota(jnp.int32, (G, N), 0)
    onehot = (jnp.broadcast_to(batch_ref[...], (G, N)) == gids).astype(_f32)
    sums = _dot_f32(onehot, y)
    cnt = jnp.sum(onehot, axis=1, keepdims=True)
    gfeat = sums / jnp.maximum(cnt, 1.0)
    gfeat = jnp.maximum(
        _dot_f32(gfeat, l1w_ref[...])
        + l1b_ref[...], 0.0)
    out_ref[...] = (_dot_f32(gfeat, l2w_ref[...])
                    + l2b_ref[...])


_tc_params = pltpu.CompilerParams(vmem_limit_bytes=100 * 1024 * 1024)

_head = pl.pallas_call(
    _head_body,
    out_shape=(jax.ShapeDtypeStruct((N, DP), _f32),
               jax.ShapeDtypeStruct((N, 1), _f32)),
    compiler_params=_tc_params)

_mid = pl.pallas_call(
    _mid_body,
    out_shape=(jax.ShapeDtypeStruct((N, DP), _f32),
               jax.ShapeDtypeStruct((N, 1), _f32)),
    compiler_params=_tc_params)

_tail = pl.pallas_call(
    _tail_body,
    out_shape=jax.ShapeDtypeStruct((G, 1), _f32),
    compiler_params=_tc_params)


CPB = 25
BLK = CPB * K


def _sc_edge_body(ht_hbm, ad_hbm, src_hbm, dst_hbm, zeros_hbm, out_hbm,
                  adv, sblk, dblk, d0, d1, r0, r1, acc, g0, g1, c0, c1):
    cid = lax.axis_index("c")
    t = lax.axis_index("s")
    Dd = (d0, d1)
    R = (r0, r1)
    Gs = (g0, g1)
    Cs = (c0, c1)

    pltpu.sync_copy(ad_hbm, adv)

    pltpu.sync_copy(zeros_hbm, r0)
    for r in range(RPT // K):
        pltpu.sync_copy(r0, acc.at[pl.ds(t * RPT + r * K, K)])
    plsc.subcore_barrier()

    ebase = (cid * NS + t) * EPT

    def load_block(iblk):
        off = ebase + iblk * BLK
        pltpu.sync_copy(src_hbm.at[pl.ds(off, BLK)], sblk)
        pltpu.sync_copy(dst_hbm.at[pl.ds(off, BLK)], dblk)

    def issue_gather(ib, b):
        cb = lax.rem(ib, CPB) * K
        for v in range(K // L):
            Dd[b][pl.ds(v * L, L)] = dblk[pl.ds(cb + v * L, L)]
        pltpu.async_copy(ht_hbm.at[sblk.at[pl.ds(cb, K)]], R[b], Gs[b])

    def step(ib, b):
        nb = 1 - b
        @pl.when(ib >= 1)
        def _():
            pltpu.make_async_copy(zeros_hbm, R[nb], Cs[nb]).wait()

        blockstart = lax.rem(ib + 1, CPB) == 0

        @pl.when(blockstart)
        def _():
            pltpu.make_async_copy(ht_hbm.at[sblk.at[pl.ds(0, K)]],
                                  R[b], Gs[b]).wait()

            @pl.when(ib + 1 < NCHUNK)
            def _():
                load_block((ib + 1) // CPB)
                issue_gather(ib + 1, nb)

        @pl.when(jnp.logical_not(blockstart))
        def _():
            issue_gather(ib + 1, nb)
            pltpu.make_async_copy(ht_hbm.at[sblk.at[pl.ds(0, K)]],
                                  R[b], Gs[b]).wait()

        rows = R[b]
        dstv = Dd[b]

        @plsc.parallel_loop(0, K // L)
        def grp(jg):
            di = dstv[pl.ds(jg * L, L)]
            advec = plsc.load_gather(adv, [di])
            rowid = jg * L + lax.iota(jnp.int32, L)
            asvec = plsc.load_gather(
                rows, [rowid, jnp.full((L,), D + 1, jnp.int32)])
            e = asvec + advec
            e = jnp.where(e >= 0.0, e, 0.2 * e)
            wvec = jnp.exp(e)
            plsc.store_scatter(rows, [rowid, jnp.full((L,), D, jnp.int32)],
                               wvec)
            for jj in range(L):
                wj = wvec[jj]
                row = jg * L + jj
                for v in range(D // L):
                    sl = pl.ds(v * L, L)
                    rows[row, sl] = rows[row, sl] * wj

        pltpu.async_copy(rows, acc.at[dstv], Cs[b], add=True)

    load_block(0)
    issue_gather(0, 0)

    def pair(ip, carry):
        step(2 * ip, 0)
        step(2 * ip + 1, 1)
        return carry
    lax.fori_loop(0, NCHUNK // 2, pair, 0)
    step(NCHUNK - 1, 0)

    pltpu.make_async_copy(zeros_hbm, R[0], Cs[0]).wait()
    plsc.subcore_barrier()

    for r in range(RPT // K):
        base = t * RPT + r * K
        pltpu.sync_copy(acc.at[pl.ds(base, K)], r0)
        pltpu.sync_copy(r0, out_hbm.at[cid, pl.ds(base, K)])


_sc_edge = pl.kernel(
    _sc_edge_body,
    out_type=jax.ShapeDtypeStruct((NC, NP, DP), _f32),
    mesh=plsc.VectorSubcoreMesh(core_axis_name="c", subcore_axis_name="s"),
    compiler_params=pltpu.CompilerParams(use_tc_tiling_on_sc=False,
                                         needs_layout_passes=False),
    scratch_types=[
        pltpu.VMEM((N,), _f32),
        pltpu.VMEM((BLK,), jnp.int32),
        pltpu.VMEM((BLK,), jnp.int32),
        pltpu.VMEM((K,), jnp.int32),
        pltpu.VMEM((K,), jnp.int32),
        pltpu.VMEM((K, DP), _f32),
        pltpu.VMEM((K, DP), _f32),
        pltpu.VMEM_SHARED((NP, DP), _f32),
        pltpu.SemaphoreType.DMA,
        pltpu.SemaphoreType.DMA,
        pltpu.SemaphoreType.DMA,
        pltpu.SemaphoreType.DMA,
    ])


def kernel(x, edge_index, batch, params):
    src = edge_index[0]
    dst = edge_index[1]
    zeros = jnp.zeros((K, DP), _f32)

    p1, p2, p3 = params["gat1"], params["gat2"], params["gat3"]
    bn1, bn2, bn3 = params["bn1"], params["bn2"], params["bn3"]

    ht, a_d = _head(x, p1["W"], p1["a_src"], p1["a_dst"])
    part = _sc_edge(ht, a_d.reshape(N), src, dst, zeros)
    ht, a_d = _mid(part, p1["b"], bn1["g"], bn1["b"],
                   p2["W"], p2["a_src"], p2["a_dst"])
    part = _sc_edge(ht, a_d.reshape(N), src, dst, zeros)
    ht, a_d = _mid(part, p2["b"], bn2["g"], bn2["b"],
                   p3["W"], p3["a_src"], p3["a_dst"])
    part = _sc_edge(ht, a_d.reshape(N), src, dst, zeros)
    return _tail(part, p3["b"], bn3["g"], bn3["b"], batch.reshape(1, N),
                 params["lin1_W"], params["lin1_b"],
                 params["lin2_W"], params["lin2_b"])
